# baseline (speedup 1.0000x reference)
import functools

import jax
import jax.numpy as jnp
from jax import lax
from jax.experimental import pallas as pl
from jax.experimental.pallas import tpu as pltpu
from jax.experimental.pallas import tpu_sc as plsc

N = 16384
K = 256
ROWS, COLS = 360, 180
P = ROWS * COLS
NC, NS, LANES = 2, 16, 16
NW = NC * NS
KB = 16
NKB = K // KB
NH = NW // NKB
HPTS = N // NH
C = 1024
NCHUNK = HPTS // C
CROWS = C // 128

_mesh = plsc.VectorSubcoreMesh(core_axis_name="c", subcore_axis_name="s")


@functools.partial(
    pl.kernel,
    out_type=jax.ShapeDtypeStruct((N, K), jnp.float32),
    mesh=_mesh,
    compiler_params=pltpu.CompilerParams(use_tc_tiling_on_sc=False,
                                         needs_layout_passes=False),
    scratch_types=[
        pltpu.VMEM((C,), jnp.float32),
        pltpu.VMEM((C,), jnp.float32),
        pltpu.VMEM((C,), jnp.int32),
        pltpu.VMEM((C,), jnp.int32),
        pltpu.VMEM((C,), jnp.float32),
        pltpu.VMEM((C,), jnp.float32),
        pltpu.VMEM((KB, C), jnp.float32),
        pltpu.VMEM((KB, C), jnp.float32),
        pltpu.VMEM((C, KB), jnp.float32),
        pltpu.SemaphoreType.DMA,
        pltpu.SemaphoreType.DMA,
    ],
)
def _sc_interp(ys, lon_in, lat_in, out, lon_v, lat_v, if_v, ic_v, wf_v, wc_v,
               gf, gc, obuf, semg, semo):
    wid = lax.axis_index("s") * NC + lax.axis_index("c")
    kblock = wid % NKB
    half = wid // NKB
    k0 = kblock * KB
    lane = lax.iota(jnp.int32, LANES)

    def chunk_body(ch, carry):
        pbase = half * HPTS + ch * C
        pltpu.sync_copy(lon_in.at[pl.ds(pbase, C)], lon_v)
        pltpu.sync_copy(lat_in.at[pl.ds(pbase, C)], lat_v)

        def idx_body(s, carry2):
            sl = pl.ds(s * LANES, LANES)
            r = lon_v[sl] + 180.0
            c = lat_v[sl] + 90.0
            fr = r.astype(jnp.int32)
            fc = c.astype(jnp.int32)
            fa = r - fr.astype(jnp.float32)
            fb = c - fc.astype(jnp.float32)
            cr = jnp.where(fa > 0.0, fr + 1, fr)
            cc = jnp.where(fb > 0.0, fc + 1, fc)
            frc = jnp.minimum(fr, ROWS - 1)
            fcc = jnp.minimum(fc, COLS - 1)
            crc = jnp.minimum(cr, ROWS - 1)
            ccc = jnp.minimum(cc, COLS - 1)
            if_v[sl] = frc * COLS + fcc
            ic_v[sl] = crc * COLS + ccc
            omb = 1.0 - fb
            wf_v[sl] = (1.0 - fa) * omb
            wc_v[sl] = fa * omb
            return carry2
        lax.fori_loop(0, C // LANES, idx_body, 0)

        copies = []
        for k in range(KB):
            copies.append(
                pltpu.async_copy(ys.at[k0 + k].at[if_v], gf.at[k], semg))
            copies.append(
                pltpu.async_copy(ys.at[k0 + k].at[ic_v], gc.at[k], semg))
        for cp in copies:
            cp.wait()

        def comb_body(s, carry2):
            sl = pl.ds(s * LANES, LANES)
            wf16 = wf_v[sl]
            wc16 = wc_v[sl]
            prow = lane + s * LANES
            for k in range(KB):
                val = wf16 * gf[k, sl] + wc16 * gc[k, sl]
                plsc.store_scatter(obuf, [prow, jnp.full((LANES,), k,
                                                         jnp.int32)], val)
            return carry2
        lax.fori_loop(0, C // LANES, comb_body, 0)

        cpo = pltpu.async_copy(
            obuf, out.at[pl.ds(pbase, C), pl.ds(k0, KB)], semo)
        cpo.wait()
        return carry

    lax.fori_loop(0, NCHUNK, chunk_body, 0)


def kernel(lonlat, Ys):
    ys = Ys.reshape(K, P)
    return _sc_interp(ys, lonlat[:, 0], lonlat[:, 1])


# speedup vs baseline: 2.6375x; 2.6375x over previous
import functools

import jax
import jax.numpy as jnp
from jax import lax
from jax.experimental import pallas as pl
from jax.experimental.pallas import tpu as pltpu
from jax.experimental.pallas import tpu_sc as plsc

N = 16384
K = 256
ROWS, COLS = 360, 180
P = ROWS * COLS
NC, NS, LANES = 2, 16, 16
NW = NC * NS
BPW = N // NW
CHUNK = 64
NCHUNK = BPW // CHUNK

RB = 8

_mesh = plsc.VectorSubcoreMesh(core_axis_name="c", subcore_axis_name="s")


def _tc_transpose_body(ys_blk, out_blk):
    x = ys_blk[...]
    x = x.reshape(K, RB * COLS)
    out_blk[...] = x.T


def _tc_transpose(Ys):
    return pl.pallas_call(
        _tc_transpose_body,
        out_shape=jax.ShapeDtypeStruct((P, K), jnp.float32),
        grid=(ROWS // RB,),
        in_specs=[pl.BlockSpec((K, RB, COLS), lambda i: (0, i, 0))],
        out_specs=pl.BlockSpec((RB * COLS, K), lambda i: (i, 0)),
    )(Ys)


@functools.partial(
    pl.kernel,
    out_type=jax.ShapeDtypeStruct((N, K), jnp.float32),
    mesh=_mesh,
    scratch_types=[
        pltpu.VMEM((CHUNK,), jnp.float32),
        pltpu.VMEM((CHUNK,), jnp.float32),
        pltpu.VMEM((2, CHUNK), jnp.int32),
        pltpu.VMEM((2, CHUNK), jnp.int32),
        pltpu.VMEM((2, CHUNK), jnp.float32),
        pltpu.VMEM((2, CHUNK), jnp.float32),
        pltpu.VMEM((2, CHUNK, K), jnp.float32),
        pltpu.VMEM((2, CHUNK, K), jnp.float32),
        pltpu.SemaphoreType.DMA,
        pltpu.SemaphoreType.DMA,
        pltpu.SemaphoreType.DMA,
    ],
)
def _sc_lookup(table, lon_in, lat_in, out, lon_v, lat_v, if_v, ic_v, wf_v,
               wc_v, bf, bc, semf, semc, semo):
    wid = lax.axis_index("s") * NC + lax.axis_index("c")
    base = wid * BPW

    def stage(ch, buf):
        cbase = base + ch * CHUNK
        pltpu.sync_copy(lon_in.at[pl.ds(cbase, CHUNK)], lon_v)
        pltpu.sync_copy(lat_in.at[pl.ds(cbase, CHUNK)], lat_v)
        for s in range(CHUNK // LANES):
            sl = pl.ds(s * LANES, LANES)
            r = lon_v[sl] + 180.0
            c = lat_v[sl] + 90.0
            fr = r.astype(jnp.int32)
            fc = c.astype(jnp.int32)
            fa = r - fr.astype(jnp.float32)
            fb = c - fc.astype(jnp.float32)
            cr = jnp.where(fa > 0.0, fr + 1, fr)
            cc = jnp.where(fb > 0.0, fc + 1, fc)
            frc = jnp.minimum(fr, ROWS - 1)
            fcc = jnp.minimum(fc, COLS - 1)
            crc = jnp.minimum(cr, ROWS - 1)
            ccc = jnp.minimum(cc, COLS - 1)
            if_v[buf, sl] = frc * COLS + fcc
            ic_v[buf, sl] = crc * COLS + ccc
            omb = 1.0 - fb
            wf_v[buf, sl] = (1.0 - fa) * omb
            wc_v[buf, sl] = fa * omb
        cpf = pltpu.async_copy(table.at[if_v.at[buf]], bf.at[buf], semf)
        cpc = pltpu.async_copy(table.at[ic_v.at[buf]], bc.at[buf], semc)
        return cpf, cpc

    def finish(ch, buf, cpf, cpc):
        cbase = base + ch * CHUNK
        cpf.wait()
        cpc.wait()

        def combine(g, carry):
            gbase = g * LANES
            wf16 = wf_v[buf, pl.ds(gbase, LANES)]
            wc16 = wc_v[buf, pl.ds(gbase, LANES)]
            for l in range(LANES):
                wfp = jnp.full((LANES,), wf16[l], jnp.float32)
                wcp = jnp.full((LANES,), wc16[l], jnp.float32)
                p = gbase + l
                for j in range(K // LANES):
                    js = pl.ds(j * LANES, LANES)
                    bf[buf, p, js] = (wfp * bf[buf, p, js]
                                      + wcp * bc[buf, p, js])
            return carry
        lax.fori_loop(0, CHUNK // LANES, combine, 0)
        return pltpu.async_copy(bf.at[buf], out.at[pl.ds(cbase, CHUNK)],
                                semo)

    cps = stage(0, 0)
    cpo_prev = None
    for ch in range(NCHUNK):
        nxt = stage(ch + 1, (ch + 1) % 2) if ch + 1 < NCHUNK else None
        if cpo_prev is not None:
            cpo_prev.wait()
        cpo_prev = finish(ch, ch % 2, *cps)
        cps = nxt
    cpo_prev.wait()


def kernel(lonlat, Ys):
    table = _tc_transpose(Ys)
    return _sc_lookup(table, lonlat[:, 0], lonlat[:, 1])


# speedup vs baseline: 9.1785x; 3.4800x over previous
import functools

import jax
import jax.numpy as jnp
from jax import lax
from jax.experimental import pallas as pl
from jax.experimental.pallas import tpu as pltpu
from jax.experimental.pallas import tpu_sc as plsc

N = 16384
K = 256
ROWS, COLS = 360, 180
P = ROWS * COLS
NC, NS, LANES = 2, 16, 16
NW = NC * NS
BPW = N // NW
CHUNK = 64
NCHUNK = BPW // CHUNK

RB = 8

_mesh = plsc.VectorSubcoreMesh(core_axis_name="c", subcore_axis_name="s")


def _tc_transpose_body(ys_blk, out_blk):
    x = ys_blk[...]
    x = x.reshape(K, RB * COLS)
    out_blk[...] = x.T


def _tc_transpose(Ys):
    return pl.pallas_call(
        _tc_transpose_body,
        out_shape=jax.ShapeDtypeStruct((P, K), jnp.float32),
        grid=(ROWS // RB,),
        in_specs=[pl.BlockSpec((K, RB, COLS), lambda i: (0, i, 0))],
        out_specs=pl.BlockSpec((RB * COLS, K), lambda i: (i, 0)),
        compiler_params=pltpu.CompilerParams(
            allow_input_fusion=(True,),
            dimension_semantics=("arbitrary",),
        ),
    )(Ys)


@functools.partial(
    pl.kernel,
    out_type=jax.ShapeDtypeStruct((N, K), jnp.float32),
    mesh=_mesh,
    scratch_types=[
        pltpu.VMEM((CHUNK,), jnp.float32),
        pltpu.VMEM((CHUNK,), jnp.float32),
        pltpu.VMEM((2, CHUNK), jnp.int32),
        pltpu.VMEM((2, CHUNK), jnp.int32),
        pltpu.VMEM((2, CHUNK), jnp.float32),
        pltpu.VMEM((2, CHUNK), jnp.float32),
        pltpu.VMEM((2, CHUNK, K), jnp.float32),
        pltpu.VMEM((2, CHUNK, K), jnp.float32),
        pltpu.SemaphoreType.DMA,
        pltpu.SemaphoreType.DMA,
        pltpu.SemaphoreType.DMA,
    ],
)
def _sc_lookup(table, lon_in, lat_in, out, lon_v, lat_v, if_v, ic_v, wf_v,
               wc_v, bf, bc, semf, semc, semo):
    wid = lax.axis_index("s") * NC + lax.axis_index("c")
    base = wid * BPW

    def stage(ch, buf):
        cbase = base + ch * CHUNK
        pltpu.sync_copy(lon_in.at[pl.ds(cbase, CHUNK)], lon_v)
        pltpu.sync_copy(lat_in.at[pl.ds(cbase, CHUNK)], lat_v)
        for s in range(CHUNK // LANES):
            sl = pl.ds(s * LANES, LANES)
            r = lon_v[sl] + 180.0
            c = lat_v[sl] + 90.0
            fr = r.astype(jnp.int32)
            fc = c.astype(jnp.int32)
            fa = r - fr.astype(jnp.float32)
            fb = c - fc.astype(jnp.float32)
            cr = jnp.where(fa > 0.0, fr + 1, fr)
            cc = jnp.where(fb > 0.0, fc + 1, fc)
            frc = jnp.minimum(fr, ROWS - 1)
            fcc = jnp.minimum(fc, COLS - 1)
            crc = jnp.minimum(cr, ROWS - 1)
            ccc = jnp.minimum(cc, COLS - 1)
            if_v[buf, sl] = fcc * ROWS + frc
            ic_v[buf, sl] = ccc * ROWS + crc
            omb = 1.0 - fb
            wf_v[buf, sl] = (1.0 - fa) * omb
            wc_v[buf, sl] = fa * omb
        cpf = pltpu.async_copy(table.at[if_v.at[buf]], bf.at[buf], semf)
        cpc = pltpu.async_copy(table.at[ic_v.at[buf]], bc.at[buf], semc)
        return cpf, cpc

    def finish(ch, buf, cpf, cpc):
        cbase = base + ch * CHUNK
        cpf.wait()
        cpc.wait()

        def combine(g, carry):
            gbase = g * LANES
            wf16 = wf_v[buf, pl.ds(gbase, LANES)]
            wc16 = wc_v[buf, pl.ds(gbase, LANES)]
            for l in range(LANES):
                wfp = jnp.full((LANES,), wf16[l], jnp.float32)
                wcp = jnp.full((LANES,), wc16[l], jnp.float32)
                p = gbase + l
                for j in range(K // LANES):
                    js = pl.ds(j * LANES, LANES)
                    bf[buf, p, js] = (wfp * bf[buf, p, js]
                                      + wcp * bc[buf, p, js])
            return carry
        lax.fori_loop(0, CHUNK // LANES, combine, 0)
        return pltpu.async_copy(bf.at[buf], out.at[pl.ds(cbase, CHUNK)],
                                semo)

    cps = stage(0, 0)
    cpo_prev = None
    for ch in range(NCHUNK):
        nxt = stage(ch + 1, (ch + 1) % 2) if ch + 1 < NCHUNK else None
        if cpo_prev is not None:
            cpo_prev.wait()
        cpo_prev = finish(ch, ch % 2, *cps)
        cps = nxt
    cpo_prev.wait()


def kernel(lonlat, Ys):
    table = jnp.transpose(Ys, (2, 1, 0)).reshape(P, K)
    return _sc_lookup(table, lonlat[:, 0], lonlat[:, 1])


# speedup vs baseline: 11.6226x; 1.2663x over previous
import functools

import jax
import jax.numpy as jnp
from jax import lax
from jax.experimental import pallas as pl
from jax.experimental.pallas import tpu as pltpu
from jax.experimental.pallas import tpu_sc as plsc

N = 16384
K = 256
ROWS, COLS = 360, 180
P = ROWS * COLS
NC, NS, LANES = 2, 16, 16
NW = NC * NS
BPW = N // NW
CHUNK = 64
NCHUNK = BPW // CHUNK

RB = 8

_mesh = plsc.VectorSubcoreMesh(core_axis_name="c", subcore_axis_name="s")


def _tc_transpose_body(ys_blk, out_blk):
    x = ys_blk[...]
    x = x.reshape(K, RB * COLS)
    out_blk[...] = x.T


def _tc_transpose(Ys):
    return pl.pallas_call(
        _tc_transpose_body,
        out_shape=jax.ShapeDtypeStruct((P, K), jnp.float32),
        grid=(ROWS // RB,),
        in_specs=[pl.BlockSpec((K, RB, COLS), lambda i: (0, i, 0))],
        out_specs=pl.BlockSpec((RB * COLS, K), lambda i: (i, 0)),
        compiler_params=pltpu.CompilerParams(
            allow_input_fusion=(True,),
            dimension_semantics=("arbitrary",),
        ),
    )(Ys)


@functools.partial(
    pl.kernel,
    out_type=jax.ShapeDtypeStruct((N, K), jnp.float32),
    mesh=_mesh,
    scratch_types=[
        pltpu.VMEM((BPW,), jnp.float32),
        pltpu.VMEM((BPW,), jnp.float32),
        pltpu.VMEM((2, CHUNK), jnp.int32),
        pltpu.VMEM((2, CHUNK), jnp.int32),
        pltpu.VMEM((2, CHUNK), jnp.float32),
        pltpu.VMEM((2, CHUNK), jnp.float32),
        pltpu.VMEM((2, CHUNK, K), jnp.float32),
        pltpu.VMEM((2, CHUNK, K), jnp.float32),
        pltpu.SemaphoreType.DMA,
        pltpu.SemaphoreType.DMA,
        pltpu.SemaphoreType.DMA,
    ],
)
def _sc_lookup(table, lon_in, lat_in, out, lon_v, lat_v, if_v, ic_v, wf_v,
               wc_v, bf, bc, semf, semc, semo):
    wid = lax.axis_index("s") * NC + lax.axis_index("c")
    base = wid * BPW
    pltpu.sync_copy(lon_in.at[pl.ds(base, BPW)], lon_v)
    pltpu.sync_copy(lat_in.at[pl.ds(base, BPW)], lat_v)

    def stage(ch, buf):
        cbase = base + ch * CHUNK
        for s in range(CHUNK // LANES):
            sl = pl.ds(s * LANES, LANES)
            ssl = pl.ds(ch * CHUNK + s * LANES, LANES)
            r = lon_v[ssl] + 180.0
            c = lat_v[ssl] + 90.0
            fr = r.astype(jnp.int32)
            fc = c.astype(jnp.int32)
            fa = r - fr.astype(jnp.float32)
            fb = c - fc.astype(jnp.float32)
            cr = jnp.where(fa > 0.0, fr + 1, fr)
            cc = jnp.where(fb > 0.0, fc + 1, fc)
            frc = jnp.minimum(fr, ROWS - 1)
            fcc = jnp.minimum(fc, COLS - 1)
            crc = jnp.minimum(cr, ROWS - 1)
            ccc = jnp.minimum(cc, COLS - 1)
            if_v[buf, sl] = fcc * ROWS + frc
            ic_v[buf, sl] = ccc * ROWS + crc
            omb = 1.0 - fb
            wf_v[buf, sl] = (1.0 - fa) * omb
            wc_v[buf, sl] = fa * omb
        pltpu.async_copy(table.at[if_v.at[buf]], bf.at[buf], semf)
        pltpu.async_copy(table.at[ic_v.at[buf]], bc.at[buf], semc)

    def drain(sem, dst):
        pltpu.make_async_copy(table.at[pl.ds(0, CHUNK)], dst, sem).wait()

    stage(0, 0)

    def chunk_body(ch, carry):
        slot = lax.rem(ch, 2)

        @pl.when(ch >= 1)
        def _():
            drain(semo, bc.at[slot])

        @pl.when(ch + 1 < NCHUNK)
        def _():
            stage(ch + 1, 1 - slot)

        drain(semf, bf.at[slot])
        drain(semc, bc.at[slot])

        def combine(g, carry2):
            gbase = g * LANES
            wf16 = wf_v[slot, pl.ds(gbase, LANES)]
            wc16 = wc_v[slot, pl.ds(gbase, LANES)]
            for l in range(LANES):
                wfp = jnp.full((LANES,), wf16[l], jnp.float32)
                wcp = jnp.full((LANES,), wc16[l], jnp.float32)
                p = gbase + l
                for j in range(K // LANES):
                    js = pl.ds(j * LANES, LANES)
                    bf[slot, p, js] = (wfp * bf[slot, p, js]
                                       + wcp * bc[slot, p, js])
            return carry2
        lax.fori_loop(0, CHUNK // LANES, combine, 0)
        cbase = base + ch * CHUNK
        pltpu.async_copy(bf.at[slot], out.at[pl.ds(cbase, CHUNK)], semo)
        return carry

    lax.fori_loop(0, NCHUNK, chunk_body, 0)
    drain(semo, bc.at[0])


def kernel(lonlat, Ys):
    table = jnp.transpose(Ys, (2, 1, 0)).reshape(P, K)
    return _sc_lookup(table, lonlat[:, 0], lonlat[:, 1])


# speedup vs baseline: 11.6874x; 1.0056x over previous
import functools

import jax
import jax.numpy as jnp
from jax import lax
from jax.experimental import pallas as pl
from jax.experimental.pallas import tpu as pltpu
from jax.experimental.pallas import tpu_sc as plsc

N = 16384
K = 256
ROWS, COLS = 360, 180
P = ROWS * COLS
NC, NS, LANES = 2, 16, 16
NW = NC * NS
BPW = N // NW
CHUNK = 64
NCHUNK = BPW // CHUNK

RB = 8

_mesh = plsc.VectorSubcoreMesh(core_axis_name="c", subcore_axis_name="s")


def _tc_transpose_body(ys_blk, out_blk):
    x = ys_blk[...]
    x = x.reshape(K, RB * COLS)
    out_blk[...] = x.T


def _tc_transpose(Ys):
    return pl.pallas_call(
        _tc_transpose_body,
        out_shape=jax.ShapeDtypeStruct((P, K), jnp.float32),
        grid=(ROWS // RB,),
        in_specs=[pl.BlockSpec((K, RB, COLS), lambda i: (0, i, 0))],
        out_specs=pl.BlockSpec((RB * COLS, K), lambda i: (i, 0)),
        compiler_params=pltpu.CompilerParams(
            allow_input_fusion=(True,),
            dimension_semantics=("arbitrary",),
        ),
    )(Ys)


@functools.partial(
    pl.kernel,
    out_type=jax.ShapeDtypeStruct((N, K), jnp.float32),
    mesh=_mesh,
    scratch_types=[
        pltpu.VMEM((BPW,), jnp.float32),
        pltpu.VMEM((BPW,), jnp.float32),
        pltpu.VMEM((3, CHUNK), jnp.int32),
        pltpu.VMEM((3, CHUNK), jnp.int32),
        pltpu.VMEM((3, CHUNK), jnp.float32),
        pltpu.VMEM((3, CHUNK), jnp.float32),
        pltpu.VMEM((3, CHUNK, K), jnp.float32),
        pltpu.VMEM((3, CHUNK, K), jnp.float32),
        pltpu.SemaphoreType.DMA,
        pltpu.SemaphoreType.DMA,
        pltpu.SemaphoreType.DMA,
    ],
)
def _sc_lookup(table, lon_in, lat_in, out, lon_v, lat_v, if_v, ic_v, wf_v,
               wc_v, bf, bc, semf, semc, semo):
    wid = lax.axis_index("s") * NC + lax.axis_index("c")
    base = wid * BPW
    pltpu.sync_copy(lon_in.at[pl.ds(base, BPW)], lon_v)
    pltpu.sync_copy(lat_in.at[pl.ds(base, BPW)], lat_v)

    def stage(ch, buf):
        cbase = base + ch * CHUNK
        for s in range(CHUNK // LANES):
            sl = pl.ds(s * LANES, LANES)
            ssl = pl.ds(ch * CHUNK + s * LANES, LANES)
            r = lon_v[ssl] + 180.0
            c = lat_v[ssl] + 90.0
            fr = r.astype(jnp.int32)
            fc = c.astype(jnp.int32)
            fa = r - fr.astype(jnp.float32)
            fb = c - fc.astype(jnp.float32)
            cr = jnp.where(fa > 0.0, fr + 1, fr)
            cc = jnp.where(fb > 0.0, fc + 1, fc)
            frc = jnp.minimum(fr, ROWS - 1)
            fcc = jnp.minimum(fc, COLS - 1)
            crc = jnp.minimum(cr, ROWS - 1)
            ccc = jnp.minimum(cc, COLS - 1)
            if_v[buf, sl] = fcc * ROWS + frc
            ic_v[buf, sl] = ccc * ROWS + crc
            omb = 1.0 - fb
            wf_v[buf, sl] = (1.0 - fa) * omb
            wc_v[buf, sl] = fa * omb
        pltpu.async_copy(table.at[if_v.at[buf]], bf.at[buf], semf)
        pltpu.async_copy(table.at[ic_v.at[buf]], bc.at[buf], semc)

    def drain(sem, dst):
        pltpu.make_async_copy(table.at[pl.ds(0, CHUNK)], dst, sem).wait()

    stage(0, 0)
    stage(1, 1)

    def chunk_body(ch, carry):
        slot = lax.rem(ch, 3)
        nslot = lax.rem(ch + 2, 3)

        @pl.when(ch >= 1)
        def _():
            drain(semo, bc.at[nslot])

        @pl.when(ch + 2 < NCHUNK)
        def _():
            stage(ch + 2, nslot)

        drain(semf, bf.at[slot])
        drain(semc, bc.at[slot])

        def combine(g, carry2):
            gbase = g * LANES
            wf16 = wf_v[slot, pl.ds(gbase, LANES)]
            wc16 = wc_v[slot, pl.ds(gbase, LANES)]
            for l in range(LANES):
                wfp = jnp.full((LANES,), wf16[l], jnp.float32)
                wcp = jnp.full((LANES,), wc16[l], jnp.float32)
                p = gbase + l
                for j in range(K // LANES):
                    js = pl.ds(j * LANES, LANES)
                    bf[slot, p, js] = (wfp * bf[slot, p, js]
                                       + wcp * bc[slot, p, js])
            return carry2
        lax.fori_loop(0, CHUNK // LANES, combine, 0)
        cbase = base + ch * CHUNK
        pltpu.async_copy(bf.at[slot], out.at[pl.ds(cbase, CHUNK)], semo)
        return carry

    lax.fori_loop(0, NCHUNK, chunk_body, 0)
    drain(semo, bc.at[0])


def kernel(lonlat, Ys):
    table = jnp.transpose(Ys, (2, 1, 0)).reshape(P, K)
    return _sc_lookup(table, lonlat[:, 0], lonlat[:, 1])
